# Initial kernel scaffold; baseline (speedup 1.0000x reference)
import jax
import jax.numpy as jnp
from jax.experimental import pallas as pl


def kernel(target_value):
    raise NotImplementedError("write your pallas kernel here")


# speedup vs baseline: 25.3987x; 25.3987x over previous
import jax
import jax.numpy as jnp
from jax.experimental import pallas as pl

_SUPPORTS_MIN = -300.0
_NUM_SUPPORTS = 601
_EPSILON = 0.001
_ROW_BLOCK = 64


def _two_hot_kernel(x_ref, o_ref):
    x = x_ref[:]
    t = jnp.sign(x) * (jnp.sqrt(jnp.abs(x) + 1.0) - 1.0 + _EPSILON * x)
    lower = jnp.clip(jnp.floor(t - _SUPPORTS_MIN), 0.0,
                     float(_NUM_SUPPORTS - 2)).astype(jnp.int32)
    upper_support = (lower + 1).astype(jnp.float32) + _SUPPORTS_MIN
    p_low = upper_support - t
    p_high = 1.0 - p_low
    rb, k = x.shape
    iota = jax.lax.broadcasted_iota(jnp.int32, (rb, k, _NUM_SUPPORTS), 2)
    lw = lower[:, :, None]
    o_ref[:] = jnp.where(
        iota == lw, p_low[:, :, None],
        jnp.where(iota == lw + 1, p_high[:, :, None], 0.0))


@jax.jit
def kernel(target_value):
    b, k = target_value.shape
    rb = _ROW_BLOCK if b % _ROW_BLOCK == 0 else 1
    return pl.pallas_call(
        _two_hot_kernel,
        grid=(b // rb,),
        in_specs=[pl.BlockSpec((rb, k), lambda i: (i, 0))],
        out_specs=pl.BlockSpec((rb, k, _NUM_SUPPORTS), lambda i: (i, 0, 0)),
        out_shape=jax.ShapeDtypeStruct((b, k, _NUM_SUPPORTS), jnp.float32),
    )(target_value)
